# baseline (speedup 1.0000x reference)
import functools

import jax
import jax.numpy as jnp
from jax import lax
from jax.experimental import pallas as pl
from jax.experimental.pallas import tpu as pltpu
from jax.experimental.pallas import tpu_sc as plsc

_B, _N, _CIN, _COUT = 8, 131072, 20, 13
_TOTAL = _B * _N
_NC, _NS = 2, 16
_NW = _NC * _NS
_PTS_W = _TOTAL // _NW
_P = 2048
_CHUNKS = _PTS_W // _P
_GROUPS = _P // 16
_IN_ROWS = _P * _CIN // 16
_OUT_ROWS = _P * _COUT // 16

_GROUPS_MAP = {
    1: [1], 2: [0], 5: [8], 6: [7], 7: [6, 12], 8: [4], 9: [5], 10: [9],
    12: [2, 3, 10, 11, 13, 14, 15, 16, 17, 18, 19],
}
_CONST_COLS = [0, 3, 4, 11]


def _sc_body(in_hbm, out_hbm, in_v, out_v):
    wid = lax.axis_index("s") * _NC + lax.axis_index("c")
    in_row0 = wid * (_PTS_W * _CIN // 16)
    out_row0 = wid * (_PTS_W * _COUT // 16)

    iota = lax.iota(jnp.int32, 16)
    in_base = [iota * _CIN + c for c in range(_CIN)]
    out_base = [iota * _COUT + t for t in range(_COUT)]
    ninf = jnp.full((16,), -jnp.inf, dtype=jnp.float32)

    def chunk_body(c, carry):
        pltpu.sync_copy(
            in_hbm.at[pl.ds(in_row0 + c * _IN_ROWS, _IN_ROWS), :], in_v)

        @plsc.parallel_loop(0, _GROUPS, unroll=8)
        def group_body(g):
            ib = g * (16 * _CIN)
            ob = g * (16 * _COUT)
            v = []
            for c_ in range(_CIN):
                w = in_base[c_] + ib
                v.append(plsc.load_gather(in_v, [w >> 4, w & 15]))
            for t, srcs in _GROUPS_MAP.items():
                acc = [v[s] for s in srcs]
                while len(acc) > 1:
                    acc = [jnp.maximum(a, b) for a, b in zip(acc[::2], acc[1::2])] + (
                        [acc[-1]] if len(acc) % 2 else [])
                o = out_base[t] + ob
                plsc.store_scatter(out_v, [o >> 4, o & 15], acc[0])
            for t in _CONST_COLS:
                o = out_base[t] + ob
                plsc.store_scatter(out_v, [o >> 4, o & 15], ninf)

        pltpu.sync_copy(
            out_v, out_hbm.at[pl.ds(out_row0 + c * _OUT_ROWS, _OUT_ROWS), :])
        return carry

    lax.fori_loop(0, _CHUNKS, chunk_body, 0)


@functools.partial(jax.jit, static_argnums=())
def kernel(logits):
    flat_in = logits.reshape(_TOTAL * _CIN // 16, 16)
    run = pl.kernel(
        _sc_body,
        out_type=jax.ShapeDtypeStruct((_TOTAL * _COUT // 16, 16), jnp.float32),
        mesh=plsc.VectorSubcoreMesh(core_axis_name="c", subcore_axis_name="s"),
        compiler_params=pltpu.CompilerParams(
            needs_layout_passes=False, use_tc_tiling_on_sc=False),
        scratch_types=[
            pltpu.VMEM((_IN_ROWS, 16), jnp.float32),
            pltpu.VMEM((_OUT_ROWS, 16), jnp.float32),
        ],
    )
    out = run(flat_in)
    return out.reshape(_B, _N, _COUT)
